# baseline (speedup 1.0000x reference)
import functools

import jax
import jax.numpy as jnp
from jax.experimental import pallas as pl
from jax.experimental.pallas import tpu as pltpu

_R = 8
_CW = 8
_SLAB = 16
_WPAD = 32
_NEG = -1e30


def _prep_kernel(q2_ref, pt_ref, wa_ref, rs_ref, cs_ref, wafull_ref, exm_ref):
    pt = pt_ref[...]
    p0 = pt[:, 0:1]
    p1 = pt[:, 1:2]
    pr0 = jnp.round(p0).astype(jnp.int32)
    pr1 = jnp.round(p1).astype(jnp.int32)
    rs = jnp.clip(pr0 - _R // 2, 0, 24 - _R)
    cs = jnp.clip(pr1 - _CW // 2, 0, 24 - _CW)
    cs_al = (cs // 8) * 8
    rs_ref[...] = rs
    cs_ref[...] = cs_al
    wafull_ref[...] = jax.lax.dot_general(
        q2_ref[...], wa_ref[...], (((1,), (1,)), ((), ())),
        preferred_element_type=jnp.float32)
    k = jax.lax.broadcasted_iota(jnp.int32, (1, _R * _SLAB), 1)
    di = k // _SLAB
    dj = k % _SLAB
    row = rs + di
    col = cs_al + dj
    off_r = row - pr0
    off_c = col - pr1
    valid = ((off_r >= -(_R // 2)) & (off_r <= (_R + 1) // 2 - 1)
             & (off_c >= -(_CW // 2)) & (off_c <= (_CW + 1) // 2 - 1)
             & (col < 24))
    rexp = -2.0 * (((row.astype(jnp.float32) - p0) / (_R // 2)) ** 2)
    cexp = -2.0 * (((col.astype(jnp.float32) - p1) / (_CW // 2)) ** 2)
    exm_ref[...] = jnp.where(valid, jnp.exp(rexp + cexp), 0.0)


def _attn_kernel(rs_sm, cs_sm, q_ref, wa_ref, c_ref, exm_ref, out_ref, *, t_len):
    b = pl.program_id(0)

    def body(t, carry):
        rs = rs_sm[b * t_len + t]
        cs = pl.multiple_of(cs_sm[b * t_len + t], 8)
        wa_s = wa_ref[0, pl.ds(rs, _R), pl.ds(cs, _SLAB), :]
        wa2 = wa_s.reshape(_R * _SLAB, wa_s.shape[-1])
        ct_row = c_ref[0, pl.ds(t, 1), :]
        a = jax.lax.dot_general(
            ct_row, wa2, (((1,), (1,)), ((), ())),
            preferred_element_type=jnp.float32)
        g = exm_ref[0, pl.ds(t, 1), :]
        am = jnp.where(g > 0.0, a, _NEG)
        e = jnp.exp(am - jnp.max(am))
        w = e * g / jnp.sum(e)
        s = q_ref[0, pl.ds(rs, _R), pl.ds(cs, _SLAB), :]
        s2 = s.reshape(_R * _SLAB, s.shape[-1])
        o = jax.lax.dot_general(
            w, s2, (((1,), (0,)), ((), ())),
            preferred_element_type=jnp.float32,
            precision=jax.lax.Precision.HIGHEST)
        out_ref[0, pl.ds(t, 1), :] = o
        return carry

    jax.lax.fori_loop(0, t_len, body, 0)


def kernel(q, c_t, W_a, W_p, V_p):
    B, C_q, H, W = q.shape
    T = c_t.shape[1]
    BT = B * T

    p_t = float(H) * jax.nn.sigmoid(jnp.tanh(c_t @ W_p.T) @ V_p.T)

    qT = q.transpose(0, 2, 3, 1)
    qT = jnp.pad(qT, ((0, 0), (0, 0), (0, _WPAD - W), (0, 0)))
    q2 = qT.reshape(B * H * _WPAD, C_q)

    rs, cs, wa_full, exm = pl.pallas_call(
        _prep_kernel,
        out_shape=(
            jax.ShapeDtypeStruct((BT, 1), jnp.int32),
            jax.ShapeDtypeStruct((BT, 1), jnp.int32),
            jax.ShapeDtypeStruct((B * H * _WPAD, C_q), jnp.float32),
            jax.ShapeDtypeStruct((BT, _R * _SLAB), jnp.float32),
        ),
    )(q2, p_t.reshape(BT, 2), W_a)

    waT = wa_full.reshape(B, H, _WPAD, C_q)
    exm3 = exm.reshape(B, T, _R * _SLAB)

    out = pl.pallas_call(
        functools.partial(_attn_kernel, t_len=T),
        grid_spec=pltpu.PrefetchScalarGridSpec(
            num_scalar_prefetch=2,
            grid=(B,),
            in_specs=[
                pl.BlockSpec((1, H, _WPAD, C_q), lambda b, rs_s, cs_s: (b, 0, 0, 0)),
                pl.BlockSpec((1, H, _WPAD, C_q), lambda b, rs_s, cs_s: (b, 0, 0, 0)),
                pl.BlockSpec((1, T, C_q), lambda b, rs_s, cs_s: (b, 0, 0)),
                pl.BlockSpec((1, T, _R * _SLAB), lambda b, rs_s, cs_s: (b, 0, 0)),
            ],
            out_specs=pl.BlockSpec((1, T, C_q), lambda b, rs_s, cs_s: (b, 0, 0)),
        ),
        out_shape=jax.ShapeDtypeStruct((B, T, C_q), jnp.float32),
        compiler_params=pltpu.CompilerParams(
            dimension_semantics=("parallel",)),
    )(rs.reshape(BT), cs.reshape(BT), qT, waT, c_t, exm3)

    return out


# speedup vs baseline: 1.0955x; 1.0955x over previous
import functools

import jax
import jax.numpy as jnp
from jax.experimental import pallas as pl
from jax.experimental.pallas import tpu as pltpu

_R = 8
_CW = 8
_SLAB = 16
_WPAD = 32
_NEG = -1e30


def _prep_kernel(q2_ref, pt_ref, wa_ref, rs_ref, cs_ref, wafull_ref, exm_ref):
    pt = pt_ref[...]
    p0 = pt[:, 0:1]
    p1 = pt[:, 1:2]
    pr0 = jnp.round(p0).astype(jnp.int32)
    pr1 = jnp.round(p1).astype(jnp.int32)
    rs = jnp.clip(pr0 - _R // 2, 0, 24 - _R)
    cs = jnp.clip(pr1 - _CW // 2, 0, 24 - _CW)
    cs_al = (cs // 8) * 8
    rs_ref[...] = rs
    cs_ref[...] = cs_al
    wafull_ref[...] = jax.lax.dot_general(
        q2_ref[...], wa_ref[...], (((1,), (1,)), ((), ())),
        preferred_element_type=jnp.float32)
    k = jax.lax.broadcasted_iota(jnp.int32, (1, _R * _SLAB), 1)
    di = k // _SLAB
    dj = k % _SLAB
    row = rs + di
    col = cs_al + dj
    off_r = row - pr0
    off_c = col - pr1
    valid = ((off_r >= -(_R // 2)) & (off_r <= (_R + 1) // 2 - 1)
             & (off_c >= -(_CW // 2)) & (off_c <= (_CW + 1) // 2 - 1)
             & (col < 24))
    rexp = -2.0 * (((row.astype(jnp.float32) - p0) / (_R // 2)) ** 2)
    cexp = -2.0 * (((col.astype(jnp.float32) - p1) / (_CW // 2)) ** 2)
    exm_ref[...] = jnp.where(valid, jnp.exp(rexp + cexp), 0.0)


def _attn_kernel(rs_sm, cs_sm, q_ref, wa_ref, c_ref, exm_ref, out_ref,
                 *, t_len, unroll):
    b = pl.program_id(0)

    def one_step(t):
        rs = rs_sm[b * t_len + t]
        cs = pl.multiple_of(cs_sm[b * t_len + t], 8)
        wa_s = wa_ref[0, pl.ds(rs, _R), pl.ds(cs, _SLAB), :]
        wa2 = wa_s.reshape(_R * _SLAB, wa_s.shape[-1])
        ct_row = c_ref[0, pl.ds(t, 1), :]
        a = jax.lax.dot_general(
            ct_row, wa2, (((1,), (1,)), ((), ())),
            preferred_element_type=jnp.float32)
        g = exm_ref[0, pl.ds(t, 1), :]
        am = jnp.where(g > 0.0, a, _NEG)
        e = jnp.exp(am - jnp.max(am))
        w = e * g / jnp.sum(e)
        s = q_ref[0, pl.ds(rs, _R), pl.ds(cs, _SLAB), :]
        s2 = s.reshape(_R * _SLAB, s.shape[-1])
        o = jax.lax.dot_general(
            w, s2, (((1,), (0,)), ((), ())),
            preferred_element_type=jnp.float32,
            precision=jax.lax.Precision.HIGHEST)
        out_ref[0, pl.ds(t, 1), :] = o

    def body(i, carry):
        for j in range(unroll):
            one_step(i * unroll + j)
        return carry

    jax.lax.fori_loop(0, t_len // unroll, body, 0)


def kernel(q, c_t, W_a, W_p, V_p):
    B, C_q, H, W = q.shape
    T = c_t.shape[1]
    BT = B * T

    p_t = float(H) * jax.nn.sigmoid(jnp.tanh(c_t @ W_p.T) @ V_p.T)

    qT = q.transpose(0, 2, 3, 1)
    qT = jnp.pad(qT, ((0, 0), (0, 0), (0, _WPAD - W), (0, 0)))
    q2 = qT.reshape(B * H * _WPAD, C_q)

    rs, cs, wa_full, exm = pl.pallas_call(
        _prep_kernel,
        out_shape=(
            jax.ShapeDtypeStruct((BT, 1), jnp.int32),
            jax.ShapeDtypeStruct((BT, 1), jnp.int32),
            jax.ShapeDtypeStruct((B * H * _WPAD, C_q), jnp.float32),
            jax.ShapeDtypeStruct((BT, _R * _SLAB), jnp.float32),
        ),
    )(q2, p_t.reshape(BT, 2), W_a)

    waT = wa_full.reshape(B, H, _WPAD, C_q)
    exm3 = exm.reshape(B, T, _R * _SLAB)

    out = pl.pallas_call(
        functools.partial(_attn_kernel, t_len=T, unroll=4),
        grid_spec=pltpu.PrefetchScalarGridSpec(
            num_scalar_prefetch=2,
            grid=(B,),
            in_specs=[
                pl.BlockSpec((1, H, _WPAD, C_q), lambda b, rs_s, cs_s: (b, 0, 0, 0)),
                pl.BlockSpec((1, H, _WPAD, C_q), lambda b, rs_s, cs_s: (b, 0, 0, 0)),
                pl.BlockSpec((1, T, C_q), lambda b, rs_s, cs_s: (b, 0, 0)),
                pl.BlockSpec((1, T, _R * _SLAB), lambda b, rs_s, cs_s: (b, 0, 0)),
            ],
            out_specs=pl.BlockSpec((1, T, C_q), lambda b, rs_s, cs_s: (b, 0, 0)),
        ),
        out_shape=jax.ShapeDtypeStruct((B, T, C_q), jnp.float32),
    )(rs.reshape(BT), cs.reshape(BT), qT, waT, c_t, exm3)

    return out


# speedup vs baseline: 2.0790x; 1.8978x over previous
import functools

import jax
import jax.numpy as jnp
from jax.experimental import pallas as pl
from jax.experimental.pallas import tpu as pltpu

_R = 8
_CW = 8
_SLAB = 16
_WPAD = 32
_G = 4
_KS = _R * _SLAB
_NEG = -1e30


def _prep_kernel(q2_ref, pt_ref, wa_ref, rs_ref, cs_ref, wafull_ref, exm_ref):
    pt = pt_ref[...]
    p0 = pt[:, 0:1]
    p1 = pt[:, 1:2]
    pr0 = jnp.round(p0).astype(jnp.int32)
    pr1 = jnp.round(p1).astype(jnp.int32)
    rs = jnp.clip(pr0 - _R // 2, 0, 24 - _R)
    cs = jnp.clip(pr1 - _CW // 2, 0, 24 - _CW)
    cs_al = (cs // 8) * 8
    rs_ref[...] = rs
    cs_ref[...] = cs_al
    wafull_ref[...] = jax.lax.dot_general(
        q2_ref[...], wa_ref[...], (((1,), (1,)), ((), ())),
        preferred_element_type=jnp.float32)
    bt = pt.shape[0]
    k = jax.lax.broadcasted_iota(jnp.int32, (1, _G * _KS), 1)
    seg = k // _KS
    kk = k % _KS
    di = kk // _SLAB
    dj = kk % _SLAB
    tmod = jax.lax.broadcasted_iota(jnp.int32, (bt, 1), 0) % _G
    row = rs + di
    col = cs_al + dj
    off_r = row - pr0
    off_c = col - pr1
    valid = ((off_r >= -(_R // 2)) & (off_r <= (_R + 1) // 2 - 1)
             & (off_c >= -(_CW // 2)) & (off_c <= (_CW + 1) // 2 - 1)
             & (col < 24) & (seg == tmod))
    rexp = -2.0 * (((row.astype(jnp.float32) - p0) / (_R // 2)) ** 2)
    cexp = -2.0 * (((col.astype(jnp.float32) - p1) / (_CW // 2)) ** 2)
    exm_ref[...] = jnp.where(valid, jnp.exp(rexp + cexp), 0.0)


def _attn_kernel(rs_sm, cs_sm, q_ref, wa_ref, c_ref, exm_ref, out_ref, *, t_len):
    b = pl.program_id(0)

    def body(i, carry):
        base = b * t_len + i * _G
        wa_slabs = []
        q_slabs = []
        for j in range(_G):
            rs = rs_sm[base + j]
            cs = pl.multiple_of(cs_sm[base + j], 8)
            wa_s = wa_ref[0, pl.ds(rs, _R), pl.ds(cs, _SLAB), :]
            wa_slabs.append(wa_s.reshape(_KS, wa_s.shape[-1]))
            q_s = q_ref[0, pl.ds(rs, _R), pl.ds(cs, _SLAB), :]
            q_slabs.append(q_s.reshape(_KS, q_s.shape[-1]))
        wa_big = jnp.concatenate(wa_slabs, axis=0)
        q_big = jnp.concatenate(q_slabs, axis=0)
        ct = c_ref[0, pl.ds(i, 1)].reshape(_G, wa_big.shape[-1])
        a = jax.lax.dot_general(
            ct, wa_big, (((1,), (1,)), ((), ())),
            preferred_element_type=jnp.float32)
        g = exm_ref[0, pl.ds(i, 1)].reshape(_G, _G * _KS)
        am = jnp.where(g > 0.0, a, _NEG)
        e = jnp.exp(am - jnp.max(am, axis=1, keepdims=True))
        w = e * g / jnp.sum(e, axis=1, keepdims=True)
        o = jax.lax.dot_general(
            w, q_big, (((1,), (0,)), ((), ())),
            preferred_element_type=jnp.float32,
            precision=jax.lax.Precision.HIGHEST)
        out_ref[0, pl.ds(i, 1)] = o.reshape(1, _G, o.shape[-1])
        return carry

    jax.lax.fori_loop(0, t_len // _G, body, 0)


def kernel(q, c_t, W_a, W_p, V_p):
    B, C_q, H, W = q.shape
    T = c_t.shape[1]
    BT = B * T

    p_t = float(H) * jax.nn.sigmoid(jnp.tanh(c_t @ W_p.T) @ V_p.T)

    qT = q.transpose(0, 2, 3, 1)
    qT = jnp.pad(qT, ((0, 0), (0, 0), (0, _WPAD - W), (0, 0)))
    q2 = qT.reshape(B * H * _WPAD, C_q)

    rs, cs, wa_full, exm = pl.pallas_call(
        _prep_kernel,
        out_shape=(
            jax.ShapeDtypeStruct((BT, 1), jnp.int32),
            jax.ShapeDtypeStruct((BT, 1), jnp.int32),
            jax.ShapeDtypeStruct((B * H * _WPAD, C_q), jnp.float32),
            jax.ShapeDtypeStruct((BT, _G * _KS), jnp.float32),
        ),
    )(q2, p_t.reshape(BT, 2), W_a)

    waT = wa_full.reshape(B, H, _WPAD, C_q)
    TG = T // _G
    c4 = c_t.reshape(B, TG, _G, C_q)
    exm4 = exm.reshape(B, TG, _G, _G * _KS)

    out = pl.pallas_call(
        functools.partial(_attn_kernel, t_len=T),
        grid_spec=pltpu.PrefetchScalarGridSpec(
            num_scalar_prefetch=2,
            grid=(B,),
            in_specs=[
                pl.BlockSpec((1, H, _WPAD, C_q), lambda b, rs_s, cs_s: (b, 0, 0, 0)),
                pl.BlockSpec((1, H, _WPAD, C_q), lambda b, rs_s, cs_s: (b, 0, 0, 0)),
                pl.BlockSpec((1, TG, _G, C_q), lambda b, rs_s, cs_s: (b, 0, 0, 0)),
                pl.BlockSpec((1, TG, _G, _G * _KS), lambda b, rs_s, cs_s: (b, 0, 0, 0)),
            ],
            out_specs=pl.BlockSpec((1, TG, _G, C_q), lambda b, rs_s, cs_s: (b, 0, 0, 0)),
        ),
        out_shape=jax.ShapeDtypeStruct((B, TG, _G, C_q), jnp.float32),
    )(rs.reshape(BT), cs.reshape(BT), qT, waT, c4, exm4)

    return out.reshape(B, T, C_q)


# speedup vs baseline: 2.5362x; 1.2199x over previous
import functools

import jax
import jax.numpy as jnp
from jax.experimental import pallas as pl
from jax.experimental.pallas import tpu as pltpu

_R = 8
_CW = 8
_SLAB = 16
_WPAD = 32
_G = 8
_KS = _R * _SLAB
_NEG = -1e30


def _prep_kernel(q2_ref, pt_ref, wa_ref, rs_ref, cs_ref, wafull_ref, exm_ref):
    pt = pt_ref[...]
    p0 = pt[:, 0:1]
    p1 = pt[:, 1:2]
    pr0 = jnp.round(p0).astype(jnp.int32)
    pr1 = jnp.round(p1).astype(jnp.int32)
    rs = jnp.clip(pr0 - _R // 2, 0, 24 - _R)
    cs = jnp.clip(pr1 - _CW // 2, 0, 24 - _CW)
    cs_al = (cs // 8) * 8
    rs_ref[...] = rs
    cs_ref[...] = cs_al
    wafull_ref[...] = jax.lax.dot_general(
        q2_ref[...], wa_ref[...], (((1,), (1,)), ((), ())),
        preferred_element_type=jnp.float32)
    bt = pt.shape[0]
    k = jax.lax.broadcasted_iota(jnp.int32, (1, _G * _KS), 1)
    seg = k // _KS
    kk = k % _KS
    di = kk // _SLAB
    dj = kk % _SLAB
    tmod = jax.lax.broadcasted_iota(jnp.int32, (bt, 1), 0) % _G
    row = rs + di
    col = cs_al + dj
    off_r = row - pr0
    off_c = col - pr1
    valid = ((off_r >= -(_R // 2)) & (off_r <= (_R + 1) // 2 - 1)
             & (off_c >= -(_CW // 2)) & (off_c <= (_CW + 1) // 2 - 1)
             & (col < 24) & (seg == tmod))
    rexp = -2.0 * (((row.astype(jnp.float32) - p0) / (_R // 2)) ** 2)
    cexp = -2.0 * (((col.astype(jnp.float32) - p1) / (_CW // 2)) ** 2)
    exm_ref[...] = jnp.where(valid, jnp.exp(rexp + cexp), 0.0)


def _attn_kernel(rs_sm, cs_sm, q_ref, wa_ref, c_ref, exm_ref, out_ref, *, t_len):
    b = pl.program_id(0)

    def body(i, carry):
        base = b * t_len + i * _G
        wa_slabs = []
        q_slabs = []
        for j in range(_G):
            rs = rs_sm[base + j]
            cs = pl.multiple_of(cs_sm[base + j], 8)
            wa_s = wa_ref[0, pl.ds(rs, _R), pl.ds(cs, _SLAB), :]
            wa_slabs.append(wa_s.reshape(_KS, wa_s.shape[-1]))
            q_s = q_ref[0, pl.ds(rs, _R), pl.ds(cs, _SLAB), :]
            q_slabs.append(q_s.reshape(_KS, q_s.shape[-1]))
        wa_big = jnp.concatenate(wa_slabs, axis=0)
        q_big = jnp.concatenate(q_slabs, axis=0)
        ct = c_ref[0, pl.ds(i, 1)].reshape(_G, wa_big.shape[-1])
        a = jax.lax.dot_general(
            ct, wa_big, (((1,), (1,)), ((), ())),
            preferred_element_type=jnp.float32)
        g = exm_ref[0, pl.ds(i, 1)].reshape(_G, _G * _KS)
        am = jnp.where(g > 0.0, a, _NEG)
        e = jnp.exp(am - jnp.max(am, axis=1, keepdims=True))
        w = e * g / jnp.sum(e, axis=1, keepdims=True)
        o = jax.lax.dot_general(
            w, q_big, (((1,), (0,)), ((), ())),
            preferred_element_type=jnp.float32,
            precision=jax.lax.Precision.HIGHEST)
        out_ref[0, pl.ds(i, 1)] = o.reshape(1, _G, o.shape[-1])
        return carry

    jax.lax.fori_loop(0, t_len // _G, body, 0)


def kernel(q, c_t, W_a, W_p, V_p):
    B, C_q, H, W = q.shape
    T = c_t.shape[1]
    BT = B * T

    p_t = float(H) * jax.nn.sigmoid(jnp.tanh(c_t @ W_p.T) @ V_p.T)

    qT = q.transpose(0, 2, 3, 1)
    qT = jnp.pad(qT, ((0, 0), (0, 0), (0, _WPAD - W), (0, 0)))
    q2 = qT.reshape(B * H * _WPAD, C_q)

    rs, cs, wa_full, exm = pl.pallas_call(
        _prep_kernel,
        out_shape=(
            jax.ShapeDtypeStruct((BT, 1), jnp.int32),
            jax.ShapeDtypeStruct((BT, 1), jnp.int32),
            jax.ShapeDtypeStruct((B * H * _WPAD, C_q), jnp.float32),
            jax.ShapeDtypeStruct((BT, _G * _KS), jnp.float32),
        ),
    )(q2, p_t.reshape(BT, 2), W_a)

    waT = wa_full.reshape(B, H, _WPAD, C_q)
    TG = T // _G
    c4 = c_t.reshape(B, TG, _G, C_q)
    exm4 = exm.reshape(B, TG, _G, _G * _KS)

    out = pl.pallas_call(
        functools.partial(_attn_kernel, t_len=T),
        grid_spec=pltpu.PrefetchScalarGridSpec(
            num_scalar_prefetch=2,
            grid=(B,),
            in_specs=[
                pl.BlockSpec((1, H, _WPAD, C_q), lambda b, rs_s, cs_s: (b, 0, 0, 0)),
                pl.BlockSpec((1, H, _WPAD, C_q), lambda b, rs_s, cs_s: (b, 0, 0, 0)),
                pl.BlockSpec((1, TG, _G, C_q), lambda b, rs_s, cs_s: (b, 0, 0, 0)),
                pl.BlockSpec((1, TG, _G, _G * _KS), lambda b, rs_s, cs_s: (b, 0, 0, 0)),
            ],
            out_specs=pl.BlockSpec((1, TG, _G, C_q), lambda b, rs_s, cs_s: (b, 0, 0, 0)),
        ),
        out_shape=jax.ShapeDtypeStruct((B, TG, _G, C_q), jnp.float32),
    )(rs.reshape(BT), cs.reshape(BT), qT, waT, c4, exm4)

    return out.reshape(B, T, C_q)


# speedup vs baseline: 2.7453x; 1.0824x over previous
import functools

import jax
import jax.numpy as jnp
from jax.experimental import pallas as pl
from jax.experimental.pallas import tpu as pltpu

_R = 8
_CW = 8
_SLAB = 16
_WPAD = 32
_G = 16
_KS = _R * _SLAB
_NEG = -1e30


def _prep_kernel(q2_ref, pt_ref, wa_ref, rs_ref, cs_ref, wafull_ref, exm_ref):
    pt = pt_ref[...]
    p0 = pt[:, 0:1]
    p1 = pt[:, 1:2]
    pr0 = jnp.round(p0).astype(jnp.int32)
    pr1 = jnp.round(p1).astype(jnp.int32)
    rs = jnp.clip(pr0 - _R // 2, 0, 24 - _R)
    cs = jnp.clip(pr1 - _CW // 2, 0, 24 - _CW)
    cs_al = (cs // 8) * 8
    rs_ref[...] = rs
    cs_ref[...] = cs_al
    wafull_ref[...] = jax.lax.dot_general(
        q2_ref[...], wa_ref[...], (((1,), (1,)), ((), ())),
        preferred_element_type=jnp.float32)
    bt = pt.shape[0]
    k = jax.lax.broadcasted_iota(jnp.int32, (1, _G * _KS), 1)
    seg = k // _KS
    kk = k % _KS
    di = kk // _SLAB
    dj = kk % _SLAB
    tmod = jax.lax.broadcasted_iota(jnp.int32, (bt, 1), 0) % _G
    row = rs + di
    col = cs_al + dj
    off_r = row - pr0
    off_c = col - pr1
    valid = ((off_r >= -(_R // 2)) & (off_r <= (_R + 1) // 2 - 1)
             & (off_c >= -(_CW // 2)) & (off_c <= (_CW + 1) // 2 - 1)
             & (col < 24) & (seg == tmod))
    rexp = -2.0 * (((row.astype(jnp.float32) - p0) / (_R // 2)) ** 2)
    cexp = -2.0 * (((col.astype(jnp.float32) - p1) / (_CW // 2)) ** 2)
    exm_ref[...] = jnp.where(valid, jnp.exp(rexp + cexp), 0.0)


def _attn_kernel(rs_sm, cs_sm, q_ref, wa_ref, c_ref, exm_ref, out_ref, *, t_len):
    b = pl.program_id(0)

    def body(i, carry):
        base = b * t_len + i * _G
        wa_slabs = []
        q_slabs = []
        for j in range(_G):
            rs = rs_sm[base + j]
            cs = pl.multiple_of(cs_sm[base + j], 8)
            wa_s = wa_ref[0, pl.ds(rs, _R), pl.ds(cs, _SLAB), :]
            wa_slabs.append(wa_s.reshape(_KS, wa_s.shape[-1]))
            q_s = q_ref[0, pl.ds(rs, _R), pl.ds(cs, _SLAB), :]
            q_slabs.append(q_s.reshape(_KS, q_s.shape[-1]))
        wa_big = jnp.concatenate(wa_slabs, axis=0)
        q_big = jnp.concatenate(q_slabs, axis=0)
        ct = c_ref[0, pl.ds(i, 1)].reshape(_G, wa_big.shape[-1])
        a = jax.lax.dot_general(
            ct, wa_big, (((1,), (1,)), ((), ())),
            preferred_element_type=jnp.float32)
        g = exm_ref[0, pl.ds(i, 1)].reshape(_G, _G * _KS)
        am = jnp.where(g > 0.0, a, _NEG)
        e = jnp.exp(am - jnp.max(am, axis=1, keepdims=True))
        w = e * g / jnp.sum(e, axis=1, keepdims=True)
        o = jax.lax.dot_general(
            w, q_big, (((1,), (0,)), ((), ())),
            preferred_element_type=jnp.float32,
            precision=jax.lax.Precision.HIGHEST)
        out_ref[0, pl.ds(i, 1)] = o.reshape(1, _G, o.shape[-1])
        return carry

    jax.lax.fori_loop(0, t_len // _G, body, 0)


def kernel(q, c_t, W_a, W_p, V_p):
    B, C_q, H, W = q.shape
    T = c_t.shape[1]
    BT = B * T

    p_t = float(H) * jax.nn.sigmoid(jnp.tanh(c_t @ W_p.T) @ V_p.T)

    qT = q.transpose(0, 2, 3, 1)
    qT = jnp.pad(qT, ((0, 0), (0, 0), (0, _WPAD - W), (0, 0)))
    q2 = qT.reshape(B * H * _WPAD, C_q)

    rs, cs, wa_full, exm = pl.pallas_call(
        _prep_kernel,
        out_shape=(
            jax.ShapeDtypeStruct((BT, 1), jnp.int32),
            jax.ShapeDtypeStruct((BT, 1), jnp.int32),
            jax.ShapeDtypeStruct((B * H * _WPAD, C_q), jnp.float32),
            jax.ShapeDtypeStruct((BT, _G * _KS), jnp.float32),
        ),
    )(q2, p_t.reshape(BT, 2), W_a)

    waT = wa_full.reshape(B, H, _WPAD, C_q)
    TG = T // _G
    c4 = c_t.reshape(B, TG, _G, C_q)
    exm4 = exm.reshape(B, TG, _G, _G * _KS)

    out = pl.pallas_call(
        functools.partial(_attn_kernel, t_len=T),
        grid_spec=pltpu.PrefetchScalarGridSpec(
            num_scalar_prefetch=2,
            grid=(B,),
            in_specs=[
                pl.BlockSpec((1, H, _WPAD, C_q), lambda b, rs_s, cs_s: (b, 0, 0, 0)),
                pl.BlockSpec((1, H, _WPAD, C_q), lambda b, rs_s, cs_s: (b, 0, 0, 0)),
                pl.BlockSpec((1, TG, _G, C_q), lambda b, rs_s, cs_s: (b, 0, 0, 0)),
                pl.BlockSpec((1, TG, _G, _G * _KS), lambda b, rs_s, cs_s: (b, 0, 0, 0)),
            ],
            out_specs=pl.BlockSpec((1, TG, _G, C_q), lambda b, rs_s, cs_s: (b, 0, 0, 0)),
        ),
        out_shape=jax.ShapeDtypeStruct((B, TG, _G, C_q), jnp.float32),
    )(rs.reshape(BT), cs.reshape(BT), qT, waT, c4, exm4)

    return out.reshape(B, T, C_q)


# speedup vs baseline: 2.7899x; 1.0162x over previous
import functools

import jax
import jax.numpy as jnp
from jax.experimental import pallas as pl
from jax.experimental.pallas import tpu as pltpu

_R = 8
_CW = 8
_SLAB = 16
_WPAD = 32
_G = 32
_KS = _R * _SLAB
_NEG = -1e30


def _prep_kernel(q2_ref, pt_ref, wa_ref, rs_ref, cs_ref, wafull_ref, exm_ref):
    pt = pt_ref[...]
    p0 = pt[:, 0:1]
    p1 = pt[:, 1:2]
    pr0 = jnp.round(p0).astype(jnp.int32)
    pr1 = jnp.round(p1).astype(jnp.int32)
    rs = jnp.clip(pr0 - _R // 2, 0, 24 - _R)
    cs = jnp.clip(pr1 - _CW // 2, 0, 24 - _CW)
    cs_al = (cs // 8) * 8
    rs_ref[...] = rs
    cs_ref[...] = cs_al
    wafull_ref[...] = jax.lax.dot_general(
        q2_ref[...], wa_ref[...], (((1,), (1,)), ((), ())),
        preferred_element_type=jnp.float32)
    bt = pt.shape[0]
    k = jax.lax.broadcasted_iota(jnp.int32, (1, _G * _KS), 1)
    seg = k // _KS
    kk = k % _KS
    di = kk // _SLAB
    dj = kk % _SLAB
    tmod = jax.lax.broadcasted_iota(jnp.int32, (bt, 1), 0) % _G
    row = rs + di
    col = cs_al + dj
    off_r = row - pr0
    off_c = col - pr1
    valid = ((off_r >= -(_R // 2)) & (off_r <= (_R + 1) // 2 - 1)
             & (off_c >= -(_CW // 2)) & (off_c <= (_CW + 1) // 2 - 1)
             & (col < 24) & (seg == tmod))
    rexp = -2.0 * (((row.astype(jnp.float32) - p0) / (_R // 2)) ** 2)
    cexp = -2.0 * (((col.astype(jnp.float32) - p1) / (_CW // 2)) ** 2)
    exm_ref[...] = jnp.where(valid, jnp.exp(rexp + cexp), 0.0)


def _attn_kernel(rs_sm, cs_sm, q_ref, wa_ref, c_ref, exm_ref, out_ref, *, t_len):
    b = pl.program_id(0)

    def body(i, carry):
        base = b * t_len + i * _G
        wa_slabs = []
        q_slabs = []
        for j in range(_G):
            rs = rs_sm[base + j]
            cs = pl.multiple_of(cs_sm[base + j], 8)
            wa_s = wa_ref[0, pl.ds(rs, _R), pl.ds(cs, _SLAB), :]
            wa_slabs.append(wa_s.reshape(_KS, wa_s.shape[-1]))
            q_s = q_ref[0, pl.ds(rs, _R), pl.ds(cs, _SLAB), :]
            q_slabs.append(q_s.reshape(_KS, q_s.shape[-1]))
        wa_big = jnp.concatenate(wa_slabs, axis=0)
        q_big = jnp.concatenate(q_slabs, axis=0)
        ct = c_ref[0, pl.ds(i, 1)].reshape(_G, wa_big.shape[-1])
        a = jax.lax.dot_general(
            ct, wa_big, (((1,), (1,)), ((), ())),
            preferred_element_type=jnp.float32)
        g = exm_ref[0, pl.ds(i, 1)].reshape(_G, _G * _KS)
        am = jnp.where(g > 0.0, a, _NEG)
        e = jnp.exp(am - jnp.max(am, axis=1, keepdims=True))
        w = e * g / jnp.sum(e, axis=1, keepdims=True)
        o = jax.lax.dot_general(
            w, q_big, (((1,), (0,)), ((), ())),
            preferred_element_type=jnp.float32,
            precision=jax.lax.Precision.HIGHEST)
        out_ref[0, pl.ds(i, 1)] = o.reshape(1, _G, o.shape[-1])
        return carry

    jax.lax.fori_loop(0, t_len // _G, body, 0)


def kernel(q, c_t, W_a, W_p, V_p):
    B, C_q, H, W = q.shape
    T = c_t.shape[1]
    BT = B * T

    p_t = float(H) * jax.nn.sigmoid(jnp.tanh(c_t @ W_p.T) @ V_p.T)

    qT = q.transpose(0, 2, 3, 1)
    qT = jnp.pad(qT, ((0, 0), (0, 0), (0, _WPAD - W), (0, 0)))
    q2 = qT.reshape(B * H * _WPAD, C_q)

    rs, cs, wa_full, exm = pl.pallas_call(
        _prep_kernel,
        out_shape=(
            jax.ShapeDtypeStruct((BT, 1), jnp.int32),
            jax.ShapeDtypeStruct((BT, 1), jnp.int32),
            jax.ShapeDtypeStruct((B * H * _WPAD, C_q), jnp.float32),
            jax.ShapeDtypeStruct((BT, _G * _KS), jnp.float32),
        ),
    )(q2, p_t.reshape(BT, 2), W_a)

    waT = wa_full.reshape(B, H, _WPAD, C_q)
    TG = T // _G
    c4 = c_t.reshape(B, TG, _G, C_q)
    exm4 = exm.reshape(B, TG, _G, _G * _KS)

    out = pl.pallas_call(
        functools.partial(_attn_kernel, t_len=T),
        grid_spec=pltpu.PrefetchScalarGridSpec(
            num_scalar_prefetch=2,
            grid=(B,),
            in_specs=[
                pl.BlockSpec((1, H, _WPAD, C_q), lambda b, rs_s, cs_s: (b, 0, 0, 0)),
                pl.BlockSpec((1, H, _WPAD, C_q), lambda b, rs_s, cs_s: (b, 0, 0, 0)),
                pl.BlockSpec((1, TG, _G, C_q), lambda b, rs_s, cs_s: (b, 0, 0, 0)),
                pl.BlockSpec((1, TG, _G, _G * _KS), lambda b, rs_s, cs_s: (b, 0, 0, 0)),
            ],
            out_specs=pl.BlockSpec((1, TG, _G, C_q), lambda b, rs_s, cs_s: (b, 0, 0, 0)),
        ),
        out_shape=jax.ShapeDtypeStruct((B, TG, _G, C_q), jnp.float32),
    )(rs.reshape(BT), cs.reshape(BT), qT, waT, c4, exm4)

    return out.reshape(B, T, C_q)
